# baseline (speedup 1.0000x reference)
import functools

import numpy as np
import jax
import jax.numpy as jnp
from jax import lax
from jax.experimental import pallas as pl
from jax.experimental.pallas import tpu as pltpu
from jax.experimental.pallas import tpu_sc as plsc

_GUMBEL_BITS = np.array([
    [1055457919, 1060849439, 3215548310, 1077986777, 1053230592, 3216131023, 1058654769, 1075020448, 1071145060, 1015105057, 1060841511, 1060840940, 1057525151, 1035105669, 1046684322, 1077845464],
    [3189265620, 1061063763, 1066986853, 1058793441, 1064650288, 1063727032, 1069667722, 3213281719, 1063576072, 3186311312, 3208004554, 1052370470, 1061090090, 1079367827, 1052595467, 1081461923],
    [1074860121, 1066223196, 1059095273, 1051662752, 3203615280, 3199130031, 1079395448, 1052541839, 3197926318, 3210949901, 1048919949, 1032067637, 1053231252, 1069895981, 1052389748, 1052348982],
    [1065556787, 3202636857, 1052581130, 3199008774, 1071984558, 1072870908, 1067903957, 1034276380, 1070680560, 1064631634, 1082461235, 3189694329, 1080925145, 1067732776, 3192194722, 3200473109],
    [1066577696, 3205015614, 3209292848, 3188467913, 1043153357, 3180983121, 1075026764, 1063292015, 3205812460, 3184807366, 1032559381, 3189187042, 1045628150, 1068207816, 1069071899, 3198805424],
    [1068991349, 3197416403, 3203365528, 3209652698, 1050602518, 3177802209, 1059051948, 1028307677, 1064141116, 3205287026, 1067500593, 1065422099, 1055744859, 1074570456, 3207806811, 3213189571],
    [1063756783, 3208226825, 1041538674, 3214712109, 1051258197, 1075857943, 1072008819, 3204451079, 1070654435, 3199305538, 3196662861, 1053119425, 1078075788, 3213287595, 1048950023, 3208011190],
    [1066355696, 3172433042, 3214122027, 3187620916, 3214404205, 1057875112, 1031072808, 3205464426, 3212001158, 1064431234, 1065412762, 1065387859, 1082571679, 1077155945, 1076241515, 1074583946],
    [1030450097, 3211137838, 3213595373, 0, 0, 0, 0, 0, 0, 0, 0, 0, 0, 0, 0, 0],
], dtype=np.uint32)
_GUMBEL = _GUMBEL_BITS.view(np.float32)

_mesh = plsc.VectorSubcoreMesh(core_axis_name="c", subcore_axis_name="s",
                               num_cores=1, num_subcores=1)


def _pick(gumbel_v, x, row0, base, n):
    best = gumbel_v[row0, :] + jnp.full((16,), x[base])
    idx = jnp.zeros((16,), jnp.int32)
    for c in range(1, n):
        s = gumbel_v[row0 + c, :] + jnp.full((16,), x[base + c])
        gt = s > best
        idx = jnp.where(gt, jnp.full((16,), c, jnp.int32), idx)
        best = jnp.where(gt, s, best)
    return idx


@functools.partial(
    pl.kernel,
    out_type=(
        jax.ShapeDtypeStruct((11,), jnp.float32),
        jax.ShapeDtypeStruct((16,), jnp.int32),
        jax.ShapeDtypeStruct((16,), jnp.int32),
        jax.ShapeDtypeStruct((16,), jnp.int32),
        jax.ShapeDtypeStruct((1,), jnp.int32),
    ),
    mesh=_mesh,
    compiler_params=pltpu.CompilerParams(needs_layout_passes=False),
    scratch_types=[
        pltpu.VMEM((4, 16), jnp.float32),
        pltpu.VMEM((8, 16), jnp.float32),
        pltpu.VMEM((16,), jnp.float32),
        pltpu.VMEM((16,), jnp.int32),
        pltpu.VMEM((16,), jnp.int32),
        pltpu.VMEM((16,), jnp.int32),
        pltpu.VMEM((16,), jnp.int32),
        pltpu.SemaphoreType.DMA,
    ],
)
def _sample_sc(fl_hbm, kl_hbm, al_hbm, ul_hbm, gumbel_hbm,
               probs_hbm, filt_hbm, kern_hbm, act_hbm, fc_hbm,
               stage_v, gumbel_v, probs_v, filt_v, kern_v, act_v, fc_v, sem):
    if True:
        cps = [
            pltpu.async_copy(fl_hbm, stage_v.at[0, pl.ds(0, 4)], sem),
            pltpu.async_copy(kl_hbm, stage_v.at[1, pl.ds(0, 2)], sem),
            pltpu.async_copy(al_hbm, stage_v.at[2, pl.ds(0, 2)], sem),
            pltpu.async_copy(ul_hbm, stage_v.at[3, pl.ds(0, 3)], sem),
            pltpu.async_copy(gumbel_hbm, gumbel_v, sem),
        ]
        for c in cps:
            c.wait()

        lane = lax.iota(jnp.int32, 16)
        maskf = lane < 4
        maskk = (lane >= 4) & (lane < 6)
        maska = (lane >= 6) & (lane < 8)
        masku = (lane >= 8) & (lane < 11)
        valid = lane < 11

        r0 = stage_v[0, :]
        s1 = stage_v[1, :].at[(lane - 4) & 15].get(mode="promise_in_bounds")
        s2 = stage_v[2, :].at[(lane - 6) & 15].get(mode="promise_in_bounds")
        s3 = stage_v[3, :].at[(lane - 8) & 15].get(mode="promise_in_bounds")
        x = jnp.where(maskf, r0, jnp.where(maskk, s1, jnp.where(maska, s2, s3)))

        neg = jnp.full((16,), -1e30, jnp.float32)
        mf = jnp.max(jnp.where(maskf, x, neg))
        mk = jnp.max(jnp.where(maskk, x, neg))
        ma = jnp.max(jnp.where(maska, x, neg))
        mu = jnp.max(jnp.where(masku, x, neg))
        gmax = jnp.where(maskf, mf, jnp.where(maskk, mk, jnp.where(maska, ma, mu)))
        e = jnp.where(valid, jnp.exp(x - gmax), 0.0)
        cs = jnp.cumsum(e)
        sf = cs[3]
        sk = cs[5] - cs[3]
        sa = cs[7] - cs[5]
        su = cs[10] - cs[7]
        gsum = jnp.where(maskf, sf, jnp.where(maskk, sk, jnp.where(maska, sa, su)))
        probs_v[...] = e / gsum

        fi = _pick(gumbel_v, x, 0, 0, 4)
        ki = _pick(gumbel_v, x, 4, 4, 2)
        ai = _pick(gumbel_v, x, 6, 6, 2)

        ubest = jnp.full((16,), float(_GUMBEL[8][0]) + x[8])
        uidx = jnp.zeros((16,), jnp.int32)
        for c in (1, 2):
            sv = jnp.full((16,), float(_GUMBEL[8][c]) + x[8 + c])
            gt = sv > ubest
            uidx = jnp.where(gt, jnp.full((16,), c, jnp.int32), uidx)
            ubest = jnp.where(gt, sv, ubest)

        filt_v[...] = fi * 32 + 16
        kern_v[...] = ki * 3 + 2
        act_v[...] = ai
        fc_v[...] = uidx * 128 + 128

        outs = [
            pltpu.async_copy(probs_v.at[pl.ds(0, 11)], probs_hbm, sem),
            pltpu.async_copy(filt_v, filt_hbm, sem),
            pltpu.async_copy(kern_v, kern_hbm, sem),
            pltpu.async_copy(act_v, act_hbm, sem),
            pltpu.async_copy(fc_v.at[pl.ds(0, 1)], fc_hbm, sem),
        ]
        for c in outs:
            c.wait()


def kernel(filter_logits, kernel_logits, activation_logits, fc_unit_logits,
           num_conv_layers):
    del num_conv_layers
    return _sample_sc(filter_logits, kernel_logits, activation_logits,
                      fc_unit_logits, jnp.asarray(_GUMBEL[:8]))


# speedup vs baseline: 1.0160x; 1.0160x over previous
import functools

import numpy as np
import jax
import jax.numpy as jnp
from jax import lax
from jax.experimental import pallas as pl
from jax.experimental.pallas import tpu as pltpu
from jax.experimental.pallas import tpu_sc as plsc

_GUMBEL_BITS = np.array([
    [1055457919, 1060849439, 3215548310, 1077986777, 1053230592, 3216131023, 1058654769, 1075020448, 1071145060, 1015105057, 1060841511, 1060840940, 1057525151, 1035105669, 1046684322, 1077845464],
    [3189265620, 1061063763, 1066986853, 1058793441, 1064650288, 1063727032, 1069667722, 3213281719, 1063576072, 3186311312, 3208004554, 1052370470, 1061090090, 1079367827, 1052595467, 1081461923],
    [1074860121, 1066223196, 1059095273, 1051662752, 3203615280, 3199130031, 1079395448, 1052541839, 3197926318, 3210949901, 1048919949, 1032067637, 1053231252, 1069895981, 1052389748, 1052348982],
    [1065556787, 3202636857, 1052581130, 3199008774, 1071984558, 1072870908, 1067903957, 1034276380, 1070680560, 1064631634, 1082461235, 3189694329, 1080925145, 1067732776, 3192194722, 3200473109],
    [1066577696, 3205015614, 3209292848, 3188467913, 1043153357, 3180983121, 1075026764, 1063292015, 3205812460, 3184807366, 1032559381, 3189187042, 1045628150, 1068207816, 1069071899, 3198805424],
    [1068991349, 3197416403, 3203365528, 3209652698, 1050602518, 3177802209, 1059051948, 1028307677, 1064141116, 3205287026, 1067500593, 1065422099, 1055744859, 1074570456, 3207806811, 3213189571],
    [1063756783, 3208226825, 1041538674, 3214712109, 1051258197, 1075857943, 1072008819, 3204451079, 1070654435, 3199305538, 3196662861, 1053119425, 1078075788, 3213287595, 1048950023, 3208011190],
    [1066355696, 3172433042, 3214122027, 3187620916, 3214404205, 1057875112, 1031072808, 3205464426, 3212001158, 1064431234, 1065412762, 1065387859, 1082571679, 1077155945, 1076241515, 1074583946],
    [1030450097, 3211137838, 3213595373, 0, 0, 0, 0, 0, 0, 0, 0, 0, 0, 0, 0, 0],
], dtype=np.uint32)
_GUMBEL = _GUMBEL_BITS.view(np.float32)

_mesh = plsc.VectorSubcoreMesh(core_axis_name="c", subcore_axis_name="s",
                               num_cores=1, num_subcores=1)


def _const_row(lane, row):
    r = jnp.full((16,), float(row[0]), jnp.float32)
    for i in range(1, 16):
        r = jnp.where(lane == i, float(row[i]), r)
    return r


def _pick(lane, x, row0, base, n):
    best = _const_row(lane, _GUMBEL[row0]) + jnp.full((16,), x[base])
    idx = jnp.zeros((16,), jnp.int32)
    for c in range(1, n):
        s = _const_row(lane, _GUMBEL[row0 + c]) + jnp.full((16,), x[base + c])
        gt = s > best
        idx = jnp.where(gt, jnp.full((16,), c, jnp.int32), idx)
        best = jnp.where(gt, s, best)
    return idx


@functools.partial(
    pl.kernel,
    out_type=(
        jax.ShapeDtypeStruct((11,), jnp.float32),
        jax.ShapeDtypeStruct((16,), jnp.int32),
        jax.ShapeDtypeStruct((16,), jnp.int32),
        jax.ShapeDtypeStruct((16,), jnp.int32),
        jax.ShapeDtypeStruct((1,), jnp.int32),
    ),
    mesh=_mesh,
    compiler_params=pltpu.CompilerParams(needs_layout_passes=False),
    scratch_types=[
        pltpu.VMEM((4, 16), jnp.float32),
        pltpu.VMEM((16,), jnp.float32),
        pltpu.VMEM((16,), jnp.int32),
        pltpu.VMEM((16,), jnp.int32),
        pltpu.VMEM((16,), jnp.int32),
        pltpu.VMEM((16,), jnp.int32),
        pltpu.SemaphoreType.DMA,
    ],
)
def _sample_sc(fl_hbm, kl_hbm, al_hbm, ul_hbm,
               probs_hbm, filt_hbm, kern_hbm, act_hbm, fc_hbm,
               stage_v, probs_v, filt_v, kern_v, act_v, fc_v, sem):
    if True:
        cps = [
            pltpu.async_copy(fl_hbm, stage_v.at[0, pl.ds(0, 4)], sem),
            pltpu.async_copy(kl_hbm, stage_v.at[1, pl.ds(0, 2)], sem),
            pltpu.async_copy(al_hbm, stage_v.at[2, pl.ds(0, 2)], sem),
            pltpu.async_copy(ul_hbm, stage_v.at[3, pl.ds(0, 3)], sem),
        ]
        for c in cps:
            c.wait()

        lane = lax.iota(jnp.int32, 16)
        maskf = lane < 4
        maskk = (lane >= 4) & (lane < 6)
        maska = (lane >= 6) & (lane < 8)
        masku = (lane >= 8) & (lane < 11)
        valid = lane < 11

        r0 = stage_v[0, :]
        s1 = stage_v[1, :].at[(lane - 4) & 15].get(mode="promise_in_bounds")
        s2 = stage_v[2, :].at[(lane - 6) & 15].get(mode="promise_in_bounds")
        s3 = stage_v[3, :].at[(lane - 8) & 15].get(mode="promise_in_bounds")
        x = jnp.where(maskf, r0, jnp.where(maskk, s1, jnp.where(maska, s2, s3)))

        neg = jnp.full((16,), -1e30, jnp.float32)
        mf = jnp.max(jnp.where(maskf, x, neg))
        mk = jnp.max(jnp.where(maskk, x, neg))
        ma = jnp.max(jnp.where(maska, x, neg))
        mu = jnp.max(jnp.where(masku, x, neg))
        gmax = jnp.where(maskf, mf, jnp.where(maskk, mk, jnp.where(maska, ma, mu)))
        e = jnp.where(valid, jnp.exp(x - gmax), 0.0)
        cs = jnp.cumsum(e)
        sf = cs[3]
        sk = cs[5] - cs[3]
        sa = cs[7] - cs[5]
        su = cs[10] - cs[7]
        gsum = jnp.where(maskf, sf, jnp.where(maskk, sk, jnp.where(maska, sa, su)))
        probs_v[...] = e / gsum

        fi = _pick(lane, x, 0, 0, 4)
        ki = _pick(lane, x, 4, 4, 2)
        ai = _pick(lane, x, 6, 6, 2)

        ubest = jnp.full((16,), float(_GUMBEL[8][0]) + x[8])
        uidx = jnp.zeros((16,), jnp.int32)
        for c in (1, 2):
            sv = jnp.full((16,), float(_GUMBEL[8][c]) + x[8 + c])
            gt = sv > ubest
            uidx = jnp.where(gt, jnp.full((16,), c, jnp.int32), uidx)
            ubest = jnp.where(gt, sv, ubest)

        filt_v[...] = fi * 32 + 16
        kern_v[...] = ki * 3 + 2
        act_v[...] = ai
        fc_v[...] = uidx * 128 + 128

        outs = [
            pltpu.async_copy(probs_v.at[pl.ds(0, 11)], probs_hbm, sem),
            pltpu.async_copy(filt_v, filt_hbm, sem),
            pltpu.async_copy(kern_v, kern_hbm, sem),
            pltpu.async_copy(act_v, act_hbm, sem),
            pltpu.async_copy(fc_v.at[pl.ds(0, 1)], fc_hbm, sem),
        ]
        for c in outs:
            c.wait()


def kernel(filter_logits, kernel_logits, activation_logits, fc_unit_logits,
           num_conv_layers):
    del num_conv_layers
    return _sample_sc(filter_logits, kernel_logits, activation_logits,
                      fc_unit_logits)


# speedup vs baseline: 1.0308x; 1.0145x over previous
import functools

import numpy as np
import jax
import jax.numpy as jnp
from jax import lax
from jax.experimental import pallas as pl
from jax.experimental.pallas import tpu as pltpu
from jax.experimental.pallas import tpu_sc as plsc

_GUMBEL_BITS = np.array([
    [1055457919, 1060849439, 3215548310, 1077986777, 1053230592, 3216131023, 1058654769, 1075020448, 1071145060, 1015105057, 1060841511, 1060840940, 1057525151, 1035105669, 1046684322, 1077845464],
    [3189265620, 1061063763, 1066986853, 1058793441, 1064650288, 1063727032, 1069667722, 3213281719, 1063576072, 3186311312, 3208004554, 1052370470, 1061090090, 1079367827, 1052595467, 1081461923],
    [1074860121, 1066223196, 1059095273, 1051662752, 3203615280, 3199130031, 1079395448, 1052541839, 3197926318, 3210949901, 1048919949, 1032067637, 1053231252, 1069895981, 1052389748, 1052348982],
    [1065556787, 3202636857, 1052581130, 3199008774, 1071984558, 1072870908, 1067903957, 1034276380, 1070680560, 1064631634, 1082461235, 3189694329, 1080925145, 1067732776, 3192194722, 3200473109],
    [1066577696, 3205015614, 3209292848, 3188467913, 1043153357, 3180983121, 1075026764, 1063292015, 3205812460, 3184807366, 1032559381, 3189187042, 1045628150, 1068207816, 1069071899, 3198805424],
    [1068991349, 3197416403, 3203365528, 3209652698, 1050602518, 3177802209, 1059051948, 1028307677, 1064141116, 3205287026, 1067500593, 1065422099, 1055744859, 1074570456, 3207806811, 3213189571],
    [1063756783, 3208226825, 1041538674, 3214712109, 1051258197, 1075857943, 1072008819, 3204451079, 1070654435, 3199305538, 3196662861, 1053119425, 1078075788, 3213287595, 1048950023, 3208011190],
    [1066355696, 3172433042, 3214122027, 3187620916, 3214404205, 1057875112, 1031072808, 3205464426, 3212001158, 1064431234, 1065412762, 1065387859, 1082571679, 1077155945, 1076241515, 1074583946],
    [1030450097, 3211137838, 3213595373, 0, 0, 0, 0, 0, 0, 0, 0, 0, 0, 0, 0, 0],
], dtype=np.uint32)
_GUMBEL = _GUMBEL_BITS.view(np.float32)

_mesh = plsc.VectorSubcoreMesh(core_axis_name="c", subcore_axis_name="s",
                               num_cores=1, num_subcores=1)


def _const_row(lane, row):
    r = jnp.full((16,), float(row[0]), jnp.float32)
    for i in range(1, 16):
        r = jnp.where(lane == i, float(row[i]), r)
    return r


def _pick(lane, x, row0, base, n):
    best = _const_row(lane, _GUMBEL[row0]) + jnp.full((16,), x[base])
    idx = jnp.zeros((16,), jnp.int32)
    for c in range(1, n):
        s = _const_row(lane, _GUMBEL[row0 + c]) + jnp.full((16,), x[base + c])
        gt = s > best
        idx = jnp.where(gt, jnp.full((16,), c, jnp.int32), idx)
        best = jnp.where(gt, s, best)
    return idx


@functools.partial(
    pl.kernel,
    out_type=(
        jax.ShapeDtypeStruct((11,), jnp.float32),
        jax.ShapeDtypeStruct((16,), jnp.int32),
        jax.ShapeDtypeStruct((16,), jnp.int32),
        jax.ShapeDtypeStruct((16,), jnp.int32),
        jax.ShapeDtypeStruct((1,), jnp.int32),
    ),
    mesh=_mesh,
    compiler_params=pltpu.CompilerParams(needs_layout_passes=False),
    scratch_types=[
        pltpu.VMEM((4, 16), jnp.float32),
        pltpu.VMEM((16,), jnp.float32),
        pltpu.VMEM((16,), jnp.int32),
        pltpu.VMEM((16,), jnp.int32),
        pltpu.VMEM((16,), jnp.int32),
        pltpu.VMEM((16,), jnp.int32),
        pltpu.SemaphoreType.DMA,
    ],
)
def _sample_sc(fl_hbm, kl_hbm, al_hbm, ul_hbm,
               probs_hbm, filt_hbm, kern_hbm, act_hbm, fc_hbm,
               stage_v, probs_v, filt_v, kern_v, act_v, fc_v, sem):
    if True:
        cps = [
            pltpu.async_copy(fl_hbm, stage_v.at[0, pl.ds(0, 4)], sem),
            pltpu.async_copy(kl_hbm, stage_v.at[1, pl.ds(0, 2)], sem),
            pltpu.async_copy(al_hbm, stage_v.at[2, pl.ds(0, 2)], sem),
            pltpu.async_copy(ul_hbm, stage_v.at[3, pl.ds(0, 3)], sem),
        ]
        for c in cps:
            c.wait()

        lane = lax.iota(jnp.int32, 16)
        maskf = lane < 4
        maskk = (lane >= 4) & (lane < 6)
        maska = (lane >= 6) & (lane < 8)
        masku = (lane >= 8) & (lane < 11)
        valid = lane < 11

        r0 = stage_v[0, :]
        s1 = stage_v[1, :].at[(lane - 4) & 15].get(mode="promise_in_bounds")
        s2 = stage_v[2, :].at[(lane - 6) & 15].get(mode="promise_in_bounds")
        s3 = stage_v[3, :].at[(lane - 8) & 15].get(mode="promise_in_bounds")
        x = jnp.where(maskf, r0, jnp.where(maskk, s1, jnp.where(maska, s2, s3)))

        neg = jnp.full((16,), -1e30, jnp.float32)
        mf = jnp.max(jnp.where(maskf, x, neg))
        mk = jnp.max(jnp.where(maskk, x, neg))
        ma = jnp.max(jnp.where(maska, x, neg))
        mu = jnp.max(jnp.where(masku, x, neg))
        gmax = jnp.where(maskf, mf, jnp.where(maskk, mk, jnp.where(maska, ma, mu)))
        e = jnp.where(valid, jnp.exp(x - gmax), 0.0)
        cs = jnp.cumsum(e)
        sf = cs[3]
        sk = cs[5] - cs[3]
        sa = cs[7] - cs[5]
        su = cs[10] - cs[7]
        gsum = jnp.where(maskf, sf, jnp.where(maskk, sk, jnp.where(maska, sa, su)))
        probs_v[...] = e / gsum
        out_probs = pltpu.async_copy(probs_v.at[pl.ds(0, 11)], probs_hbm, sem)

        fi = _pick(lane, x, 0, 0, 4)
        filt_v[...] = fi * 32 + 16
        out_filt = pltpu.async_copy(filt_v, filt_hbm, sem)
        ki = _pick(lane, x, 4, 4, 2)
        kern_v[...] = ki * 3 + 2
        out_kern = pltpu.async_copy(kern_v, kern_hbm, sem)
        ai = _pick(lane, x, 6, 6, 2)
        act_v[...] = ai
        out_act = pltpu.async_copy(act_v, act_hbm, sem)

        ubest = jnp.full((16,), float(_GUMBEL[8][0]) + x[8])
        uidx = jnp.zeros((16,), jnp.int32)
        for c in (1, 2):
            sv = jnp.full((16,), float(_GUMBEL[8][c]) + x[8 + c])
            gt = sv > ubest
            uidx = jnp.where(gt, jnp.full((16,), c, jnp.int32), uidx)
            ubest = jnp.where(gt, sv, ubest)

        fc_v[...] = uidx * 128 + 128
        out_fc = pltpu.async_copy(fc_v.at[pl.ds(0, 1)], fc_hbm, sem)

        for c in (out_probs, out_filt, out_kern, out_act, out_fc):
            c.wait()


def kernel(filter_logits, kernel_logits, activation_logits, fc_unit_logits,
           num_conv_layers):
    del num_conv_layers
    return _sample_sc(filter_logits, kernel_logits, activation_logits,
                      fc_unit_logits)


# speedup vs baseline: 1.0309x; 1.0001x over previous
import functools

import numpy as np
import jax
import jax.numpy as jnp
from jax import lax
from jax.experimental import pallas as pl
from jax.experimental.pallas import tpu as pltpu
from jax.experimental.pallas import tpu_sc as plsc

_GUMBEL_BITS = np.array([
    [1055457919, 1060849439, 3215548310, 1077986777, 1053230592, 3216131023, 1058654769, 1075020448, 1071145060, 1015105057, 1060841511, 1060840940, 1057525151, 1035105669, 1046684322, 1077845464],
    [3189265620, 1061063763, 1066986853, 1058793441, 1064650288, 1063727032, 1069667722, 3213281719, 1063576072, 3186311312, 3208004554, 1052370470, 1061090090, 1079367827, 1052595467, 1081461923],
    [1074860121, 1066223196, 1059095273, 1051662752, 3203615280, 3199130031, 1079395448, 1052541839, 3197926318, 3210949901, 1048919949, 1032067637, 1053231252, 1069895981, 1052389748, 1052348982],
    [1065556787, 3202636857, 1052581130, 3199008774, 1071984558, 1072870908, 1067903957, 1034276380, 1070680560, 1064631634, 1082461235, 3189694329, 1080925145, 1067732776, 3192194722, 3200473109],
    [1066577696, 3205015614, 3209292848, 3188467913, 1043153357, 3180983121, 1075026764, 1063292015, 3205812460, 3184807366, 1032559381, 3189187042, 1045628150, 1068207816, 1069071899, 3198805424],
    [1068991349, 3197416403, 3203365528, 3209652698, 1050602518, 3177802209, 1059051948, 1028307677, 1064141116, 3205287026, 1067500593, 1065422099, 1055744859, 1074570456, 3207806811, 3213189571],
    [1063756783, 3208226825, 1041538674, 3214712109, 1051258197, 1075857943, 1072008819, 3204451079, 1070654435, 3199305538, 3196662861, 1053119425, 1078075788, 3213287595, 1048950023, 3208011190],
    [1066355696, 3172433042, 3214122027, 3187620916, 3214404205, 1057875112, 1031072808, 3205464426, 3212001158, 1064431234, 1065412762, 1065387859, 1082571679, 1077155945, 1076241515, 1074583946],
    [1030450097, 3211137838, 3213595373, 0, 0, 0, 0, 0, 0, 0, 0, 0, 0, 0, 0, 0],
], dtype=np.uint32)
_GUMBEL = _GUMBEL_BITS.view(np.float32)

_mesh = plsc.VectorSubcoreMesh(core_axis_name="c", subcore_axis_name="s",
                               num_cores=1, num_subcores=1)


def _const_row(lane, row):
    r = jnp.full((16,), float(row[0]), jnp.float32)
    for i in range(1, 16):
        r = jnp.where(lane == i, float(row[i]), r)
    return r


def _pick(lane, x, row0, base, n):
    best = _const_row(lane, _GUMBEL[row0]) + jnp.full((16,), x[base])
    idx = jnp.zeros((16,), jnp.int32)
    for c in range(1, n):
        s = _const_row(lane, _GUMBEL[row0 + c]) + jnp.full((16,), x[base + c])
        gt = s > best
        idx = jnp.where(gt, jnp.full((16,), c, jnp.int32), idx)
        best = jnp.where(gt, s, best)
    return idx


@functools.partial(
    pl.kernel,
    out_type=(
        jax.ShapeDtypeStruct((11,), jnp.float32),
        jax.ShapeDtypeStruct((16,), jnp.int32),
        jax.ShapeDtypeStruct((16,), jnp.int32),
        jax.ShapeDtypeStruct((16,), jnp.int32),
        jax.ShapeDtypeStruct((1,), jnp.int32),
    ),
    mesh=_mesh,
    compiler_params=pltpu.CompilerParams(needs_layout_passes=False),
    scratch_types=[
        pltpu.VMEM((4, 16), jnp.float32),
        pltpu.VMEM((16,), jnp.float32),
        pltpu.VMEM((16,), jnp.int32),
        pltpu.VMEM((16,), jnp.int32),
        pltpu.VMEM((16,), jnp.int32),
        pltpu.VMEM((16,), jnp.int32),
        pltpu.SemaphoreType.DMA,
    ],
)
def _sample_sc(fl_hbm, kl_hbm, al_hbm, ul_hbm,
               probs_hbm, filt_hbm, kern_hbm, act_hbm, fc_hbm,
               stage_v, probs_v, filt_v, kern_v, act_v, fc_v, sem):
    cps = [
        pltpu.async_copy(fl_hbm, stage_v.at[0, pl.ds(0, 4)], sem),
        pltpu.async_copy(kl_hbm, stage_v.at[1, pl.ds(0, 2)], sem),
        pltpu.async_copy(al_hbm, stage_v.at[2, pl.ds(0, 2)], sem),
        pltpu.async_copy(ul_hbm, stage_v.at[3, pl.ds(0, 3)], sem),
    ]
    for c in cps:
        c.wait()

    lane = lax.iota(jnp.int32, 16)
    maskf = lane < 4
    maskk = (lane >= 4) & (lane < 6)
    maska = (lane >= 6) & (lane < 8)
    masku = (lane >= 8) & (lane < 11)
    valid = lane < 11

    r0 = stage_v[0, :]
    s1 = stage_v[1, :].at[(lane - 4) & 15].get(mode="promise_in_bounds")
    s2 = stage_v[2, :].at[(lane - 6) & 15].get(mode="promise_in_bounds")
    s3 = stage_v[3, :].at[(lane - 8) & 15].get(mode="promise_in_bounds")
    x = jnp.where(maskf, r0, jnp.where(maskk, s1, jnp.where(maska, s2, s3)))

    neg = jnp.full((16,), -1e30, jnp.float32)
    mf = jnp.max(jnp.where(maskf, x, neg))
    mk = jnp.max(jnp.where(maskk, x, neg))
    ma = jnp.max(jnp.where(maska, x, neg))
    mu = jnp.max(jnp.where(masku, x, neg))
    gmax = jnp.where(maskf, mf, jnp.where(maskk, mk, jnp.where(maska, ma, mu)))
    e = jnp.where(valid, jnp.exp(x - gmax), 0.0)
    cs = jnp.cumsum(e)
    sf = cs[3]
    sk = cs[5] - cs[3]
    sa = cs[7] - cs[5]
    su = cs[10] - cs[7]
    gsum = jnp.where(maskf, sf, jnp.where(maskk, sk, jnp.where(maska, sa, su)))
    probs_v[...] = e / gsum
    out_probs = pltpu.async_copy(probs_v.at[pl.ds(0, 11)], probs_hbm, sem)

    fi = _pick(lane, x, 0, 0, 4)
    filt_v[...] = fi * 32 + 16
    out_filt = pltpu.async_copy(filt_v, filt_hbm, sem)
    ki = _pick(lane, x, 4, 4, 2)
    kern_v[...] = ki * 3 + 2
    out_kern = pltpu.async_copy(kern_v, kern_hbm, sem)
    ai = _pick(lane, x, 6, 6, 2)
    act_v[...] = ai
    out_act = pltpu.async_copy(act_v, act_hbm, sem)

    ubest = jnp.full((16,), float(_GUMBEL[8][0]) + x[8])
    uidx = jnp.zeros((16,), jnp.int32)
    for c in (1, 2):
        sv = jnp.full((16,), float(_GUMBEL[8][c]) + x[8 + c])
        gt = sv > ubest
        uidx = jnp.where(gt, jnp.full((16,), c, jnp.int32), uidx)
        ubest = jnp.where(gt, sv, ubest)

    fc_v[...] = uidx * 128 + 128
    out_fc = pltpu.async_copy(fc_v.at[pl.ds(0, 1)], fc_hbm, sem)

    for c in (out_probs, out_filt, out_kern, out_act, out_fc):
        c.wait()


def kernel(filter_logits, kernel_logits, activation_logits, fc_unit_logits,
           num_conv_layers):
    del num_conv_layers
    return _sample_sc(filter_logits, kernel_logits, activation_logits,
                      fc_unit_logits)
